# baseline (speedup 1.0000x reference)
import functools

import jax
import jax.numpy as jnp
from jax import lax
from jax.experimental import pallas as pl
from jax.experimental.pallas import tpu as pltpu
from jax.experimental.pallas import tpu_sc as plsc

_N_STIMULI = 20
_N_DIM = 3
_N_REF = 8
_BATCH = 16384
_BETA = 10.0
_GAMMA = 0.001

_ROWS = _N_STIMULI + 1
_SSTRIDE = 32

_NC = 2
_NS = 16
_NW = _NC * _NS
_BPW = _BATCH // _NW
_L = 16
_GROUPS = _BPW // _L
_NCOL = 1 + _N_REF


def _vexp(x):
    x = jnp.maximum(x, -60.0)
    t = x * 1.4426950408889634
    ki = t.astype(jnp.int32)
    ki = ki - jnp.where(t < ki.astype(jnp.float32), 1, 0)
    g = (t - ki.astype(jnp.float32)) * 0.6931471805599453
    p = jnp.float32(1.0 / 720.0)
    for c in (1.0 / 120.0, 1.0 / 24.0, 1.0 / 6.0, 0.5, 1.0, 1.0):
        p = p * g + c
    scale = plsc.bitcast(lax.shift_left(ki + 127, 23), jnp.float32)
    return scale * p


def _vsqrt(x):
    y = plsc.bitcast(
        lax.shift_right_logical(plsc.bitcast(x, jnp.int32), 1) + 0x1FBD1DF5,
        jnp.float32,
    )
    y = 0.5 * (y + x / y)
    y = 0.5 * (y + x / y)
    return y


_sc_mesh = plsc.VectorSubcoreMesh(
    core_axis_name="c", subcore_axis_name="s", num_cores=_NC, num_subcores=_NS
)


@functools.partial(
    pl.kernel,
    mesh=_sc_mesh,
    compiler_params=pltpu.CompilerParams(
        needs_layout_passes=False,
        disable_bounds_checks=True,
        disable_semaphore_checks=True,
        skip_device_barrier=True,
    ),
    out_type=jax.ShapeDtypeStruct((_BATCH,), jnp.float32),
    scratch_types=[
        pltpu.VMEM((_BPW * _NCOL,), jnp.int32),
        pltpu.VMEM((2 * _L * _N_DIM,), jnp.float32),
        pltpu.VMEM((_ROWS * _SSTRIDE,), jnp.float32),
        pltpu.VMEM((_BPW,), jnp.float32),
        pltpu.SemaphoreType.DMA,
    ],
)
def _rank_sc(stim_hbm, tbl_hbm, out_hbm, stim_v, t_v, s_v, out_v, sem):
    wid = lax.axis_index("s") * _NC + lax.axis_index("c")
    base = wid * _BPW
    lanes = lax.iota(jnp.int32, _L)

    stim_dma = pltpu.async_copy(
        stim_hbm.at[pl.ds(base * _NCOL, _BPW * _NCOL)], stim_v, sem
    )
    pltpu.sync_copy(tbl_hbm, t_v.at[pl.ds(0, _ROWS * _N_DIM)])

    tcoord = [
        [plsc.load_gather(t_v, [(lanes + g * _L) * _N_DIM + d]) for g in range(2)]
        for d in range(_N_DIM)
    ]

    @plsc.parallel_loop(0, _ROWS)
    def _table(i):
        in_g0 = i < _L
        sel = jnp.broadcast_to(jnp.where(in_g0, i, i - _L), (_L,))
        ti = [
            jnp.where(
                in_g0,
                jnp.take_along_axis(
                    tcoord[d][0], sel, axis=0, mode="promise_in_bounds"
                ),
                jnp.take_along_axis(
                    tcoord[d][1], sel, axis=0, mode="promise_in_bounds"
                ),
            )
            for d in range(_N_DIM)
        ]
        for g in range(2):
            d2 = jnp.zeros((_L,), jnp.float32)
            for d in range(_N_DIM):
                diff = ti[d] - tcoord[d][g]
                d2 = d2 + diff * diff
            s = _vexp(-_BETA * _vsqrt(d2 + 1e-12)) + _GAMMA
            if g == 0:
                s = jnp.where(lanes == 0, 0.0, s)
            s_v[pl.ds(i * _SSTRIDE + g * _L, _L)] = s

    stim_dma.wait()

    @plsc.parallel_loop(0, _GROUPS, unroll=4)
    def _rank(i):
        rowbase = (lanes + (i * _L)) * _NCOL
        q = plsc.load_gather(stim_v, [rowbase])
        q_off = q * _SSTRIDE
        sk = []
        for k in range(_N_REF):
            r = plsc.load_gather(stim_v, [rowbase + (k + 1)])
            sk.append(plsc.load_gather(s_v, [q_off + r]))
        total = sk[0]
        for k in range(1, _N_REF):
            total = total + sk[k]
        p0 = sk[0] / jnp.maximum(total, 1e-30)
        p1 = sk[1] / jnp.maximum(total - sk[0], 1e-30)
        out_v[pl.ds(i * _L, _L)] = p0 * p1

    pltpu.sync_copy(out_v, out_hbm.at[pl.ds(base, _BPW)])


def kernel(stimulus_set, percept_table):
    return _rank_sc(
        stimulus_set.reshape(_BATCH * _NCOL),
        percept_table.reshape(_ROWS * _N_DIM),
    )


# speedup vs baseline: 1.0067x; 1.0067x over previous
import functools

import jax
import jax.numpy as jnp
from jax import lax
from jax.experimental import pallas as pl
from jax.experimental.pallas import tpu as pltpu
from jax.experimental.pallas import tpu_sc as plsc

_N_STIMULI = 20
_N_DIM = 3
_N_REF = 8
_BATCH = 16384
_BETA = 10.0
_GAMMA = 0.001

_ROWS = _N_STIMULI + 1
_SSTRIDE = 32

_NC = 2
_NS = 16
_NW = _NC * _NS
_BPW = _BATCH // _NW
_L = 16
_GROUPS = _BPW // _L
_NCOL = 1 + _N_REF


def _vexp(x):
    x = jnp.maximum(x, -60.0)
    t = x * 1.4426950408889634
    ki = t.astype(jnp.int32)
    ki = ki - jnp.where(t < ki.astype(jnp.float32), 1, 0)
    g = (t - ki.astype(jnp.float32)) * 0.6931471805599453
    p = jnp.float32(1.0 / 720.0)
    for c in (1.0 / 120.0, 1.0 / 24.0, 1.0 / 6.0, 0.5, 1.0, 1.0):
        p = p * g + c
    scale = plsc.bitcast(lax.shift_left(ki + 127, 23), jnp.float32)
    return scale * p


def _vsqrt(x):
    y = plsc.bitcast(
        lax.shift_right_logical(plsc.bitcast(x, jnp.int32), 1) + 0x1FBD1DF5,
        jnp.float32,
    )
    y = 0.5 * (y + x / y)
    y = 0.5 * (y + x / y)
    return y


_sc_mesh = plsc.VectorSubcoreMesh(
    core_axis_name="c", subcore_axis_name="s", num_cores=_NC, num_subcores=_NS
)


@functools.partial(
    pl.kernel,
    mesh=_sc_mesh,
    compiler_params=pltpu.CompilerParams(
        needs_layout_passes=False,
        disable_bounds_checks=True,
        disable_semaphore_checks=True,
        skip_device_barrier=True,
    ),
    out_type=jax.ShapeDtypeStruct((_BATCH,), jnp.float32),
    scratch_types=[
        pltpu.VMEM((_BPW * _NCOL,), jnp.int32),
        pltpu.VMEM((2 * _L * _N_DIM,), jnp.float32),
        pltpu.VMEM((_ROWS * _SSTRIDE,), jnp.float32),
        pltpu.VMEM((_BPW,), jnp.float32),
        pltpu.SemaphoreType.DMA,
    ],
)
def _rank_sc(stim_hbm, tbl_hbm, out_hbm, stim_v, t_v, s_v, out_v, sem):
    wid = lax.axis_index("s") * _NC + lax.axis_index("c")
    base = wid * _BPW
    lanes = lax.iota(jnp.int32, _L)

    stim_dma = pltpu.async_copy(
        stim_hbm.at[pl.ds(base * _NCOL, _BPW * _NCOL)], stim_v, sem
    )
    pltpu.sync_copy(tbl_hbm, t_v.at[pl.ds(0, _ROWS * _N_DIM)])

    tcoord = [
        [plsc.load_gather(t_v, [(lanes + g * _L) * _N_DIM + d]) for g in range(2)]
        for d in range(_N_DIM)
    ]

    @plsc.parallel_loop(0, _ROWS, unroll=3)
    def _table(i):
        in_g0 = i < _L
        sel = jnp.broadcast_to(jnp.where(in_g0, i, i - _L), (_L,))
        ti = [
            jnp.where(
                in_g0,
                jnp.take_along_axis(
                    tcoord[d][0], sel, axis=0, mode="promise_in_bounds"
                ),
                jnp.take_along_axis(
                    tcoord[d][1], sel, axis=0, mode="promise_in_bounds"
                ),
            )
            for d in range(_N_DIM)
        ]
        for g in range(2):
            d2 = jnp.zeros((_L,), jnp.float32)
            for d in range(_N_DIM):
                diff = ti[d] - tcoord[d][g]
                d2 = d2 + diff * diff
            s = _vexp(-_BETA * _vsqrt(d2 + 1e-12)) + _GAMMA
            if g == 0:
                s = jnp.where(lanes == 0, 0.0, s)
            s_v[pl.ds(i * _SSTRIDE + g * _L, _L)] = s

    stim_dma.wait()

    @plsc.parallel_loop(0, _GROUPS, unroll=2)
    def _rank(i):
        rowbase = (lanes + (i * _L)) * _NCOL
        q = plsc.load_gather(stim_v, [rowbase])
        q_off = q * _SSTRIDE
        sk = []
        for k in range(_N_REF):
            r = plsc.load_gather(stim_v, [rowbase + (k + 1)])
            sk.append(plsc.load_gather(s_v, [q_off + r]))
        total = sk[0]
        for k in range(1, _N_REF):
            total = total + sk[k]
        p0 = sk[0] / jnp.maximum(total, 1e-30)
        p1 = sk[1] / jnp.maximum(total - sk[0], 1e-30)
        out_v[pl.ds(i * _L, _L)] = p0 * p1

    pltpu.sync_copy(out_v, out_hbm.at[pl.ds(base, _BPW)])


def kernel(stimulus_set, percept_table):
    return _rank_sc(
        stimulus_set.reshape(_BATCH * _NCOL),
        percept_table.reshape(_ROWS * _N_DIM),
    )
